# baseline (speedup 1.0000x reference)
import jax
import jax.numpy as jnp
from jax.experimental import pallas as pl
from jax.experimental.pallas import tpu as pltpu

B, V, F = 16384, 130, 128
C = V * 3
TB = 512
GRID = B // TB
SCALE = 1.0 / (B * 3 * F)


def _edge_partial(co, cg, s):
    k = 3 * s
    n = C - k - 2
    do = co[:, k:] - co[:, :C - k]
    dg = cg[:, k:] - cg[:, :C - k]
    so = do * do
    sg = dg * dg
    to = so[:, :n] + so[:, 1:n + 1] + so[:, 2:n + 2]
    tg = sg[:, :n] + sg[:, 1:n + 1] + sg[:, 2:n + 2]
    diff = jnp.abs(jnp.sqrt(to) - jnp.sqrt(tg))
    lane = jax.lax.broadcasted_iota(jnp.int32, (1, n), 1)
    valid = lane % 3 == 0
    if s == 1:
        w = jnp.where(valid, jnp.where((lane == 0) | (lane == 3 * (F + 1 - s)), 1.0, 2.0), 0.0)
    else:
        w = jnp.where(valid, 1.0, 0.0)
    return jnp.sum(diff * (w * SCALE))


def _loss_kernel(out_ref, gt_ref, acc_ref):
    co = out_ref[...]
    cg = gt_ref[...]
    partial = _edge_partial(co, cg, 1) + _edge_partial(co, cg, 2)

    @pl.when(pl.program_id(0) == 0)
    def _():
        acc_ref[0, 0] = 0.0

    acc_ref[0, 0] += partial


def kernel(coord_out, coord_gt):
    co = coord_out.reshape(B, C)
    cg = coord_gt.reshape(B, C)
    out = pl.pallas_call(
        _loss_kernel,
        grid=(GRID,),
        in_specs=[
            pl.BlockSpec((TB, C), lambda i: (i, 0)),
            pl.BlockSpec((TB, C), lambda i: (i, 0)),
        ],
        out_specs=pl.BlockSpec(memory_space=pltpu.SMEM),
        out_shape=jax.ShapeDtypeStruct((1, 1), jnp.float32),
    )(co, cg)
    return out[0, 0]


# speedup vs baseline: 7.2843x; 7.2843x over previous
import jax
import jax.numpy as jnp
from jax.experimental import pallas as pl
from jax.experimental.pallas import tpu as pltpu

B, V, F = 16384, 130, 128
TBL = 2048
GRID = B // TBL
SCALE = 1.0 / (B * 3 * F)


def _loss_kernel(out_ref, gt_ref, acc_ref):
    co = out_ref[...]
    cg = gt_ref[...]

    def lengths(c, s):
        d = c[:, s:, :] - c[:, : V - s, :]
        return jnp.sqrt(d[0] * d[0] + d[1] * d[1] + d[2] * d[2])

    diff1 = jnp.abs(lengths(co, 1) - lengths(cg, 1))
    diff2 = jnp.abs(lengths(co, 2) - lengths(cg, 2))
    row = jax.lax.broadcasted_iota(jnp.int32, (V - 1, 1), 0)
    w1 = jnp.where((row == 0) | (row == V - 2), SCALE, 2.0 * SCALE)
    partial = jnp.sum(diff1 * w1) + jnp.sum(diff2) * SCALE

    @pl.when(pl.program_id(0) == 0)
    def _():
        acc_ref[0, 0] = 0.0

    acc_ref[0, 0] += partial


def kernel(coord_out, coord_gt):
    co = jnp.transpose(coord_out, (2, 1, 0))
    cg = jnp.transpose(coord_gt, (2, 1, 0))
    out = pl.pallas_call(
        _loss_kernel,
        grid=(GRID,),
        in_specs=[
            pl.BlockSpec((3, V, TBL), lambda i: (0, 0, i)),
            pl.BlockSpec((3, V, TBL), lambda i: (0, 0, i)),
        ],
        out_specs=pl.BlockSpec(memory_space=pltpu.SMEM),
        out_shape=jax.ShapeDtypeStruct((1, 1), jnp.float32),
    )(co, cg)
    return out[0, 0]
